# Initial kernel scaffold; baseline (speedup 1.0000x reference)
import jax
import jax.numpy as jnp
from jax.experimental import pallas as pl


def kernel(features, coors, W1, g1, b1, rm1, rv1, W2, g2, b2, rm2, rv2, W3, b3):
    raise NotImplementedError("write your pallas kernel here")


# speedup vs baseline: 134.0057x; 134.0057x over previous
import jax
import jax.numpy as jnp
from jax.experimental import pallas as pl
from jax.experimental.pallas import tpu as pltpu

_N = 8192
_BI = 512
_BJ = 2048
_R = 3
_EPS = 1e-5


def _cross(a, b):
    return (a[1] * b[2] - a[2] * b[1],
            a[2] * b[0] - a[0] * b[2],
            a[0] * b[1] - a[1] * b[0])


def _rcp(x):
    r = 1.0 / x
    r = r * (2.0 - x * r)
    r = r * (2.0 - x * r)
    return r


def _rsqrt(x):
    y = jax.lax.rsqrt(x)
    y = y * (1.5 - 0.5 * x * y * y)
    y = y * (1.5 - 0.5 * x * y * y)
    return y


def _sqrt(x):
    xs = jnp.maximum(x, 1e-38)
    return xs * _rsqrt(xs)


def _body(feat_t, coors_col, coors_row, w1, g1, b1, rm1, rv1,
          w2, g2, b2, rm2, rv2, w3, b3, out_ref):
    i = pl.program_id(0)
    i0 = i * _BI

    px_all = feat_t[0:1, :]
    py_all = feat_t[1:2, :]
    pz_all = feat_t[2:3, :]
    xt = jnp.concatenate([
        jnp.ones_like(px_all), px_all, py_all, pz_all,
        px_all * px_all, px_all * py_all, px_all * pz_all,
        py_all * py_all, py_all * pz_all, pz_all * pz_all,
    ], axis=0)
    xt0 = xt.astype(jnp.bfloat16)

    bi = coors_row[0:1, pl.ds(i0, _BI)]
    yi = coors_row[1:2, pl.ds(i0, _BI)]
    xi = coors_row[2:3, pl.ds(i0, _BI)]

    bj = coors_col[:, 0:1]
    yj = coors_col[:, 1:2]
    xj = coors_col[:, 2:3]
    m = ((bj == bi)
         & (jnp.abs(yj - yi) <= _R)
         & (jnp.abs(xj - xi) <= _R)).astype(jnp.bfloat16)
    acc = jax.lax.dot_general(
        xt0, m, (((1,), (0,)), ((), ())),
        preferred_element_type=jnp.float32)

    pxi = feat_t[0:1, pl.ds(i0, _BI)]
    pyi = feat_t[1:2, pl.ds(i0, _BI)]
    pzi = feat_t[2:3, pl.ds(i0, _BI)]
    _b = lambda v: v.astype(jnp.bfloat16).astype(jnp.float32)
    n = acc[0:1] - 1.0
    s1x = acc[1:2] - _b(pxi)
    s1y = acc[2:3] - _b(pyi)
    s1z = acc[3:4] - _b(pzi)
    sxx = acc[4:5] - _b(pxi * pxi)
    sxy = acc[5:6] - _b(pxi * pyi)
    sxz = acc[6:7] - _b(pxi * pzi)
    syy = acc[7:8] - _b(pyi * pyi)
    syz = acc[8:9] - _b(pyi * pzi)
    szz = acc[9:10] - _b(pzi * pzi)

    nrcp = _rcp(jnp.maximum(n, 1.0))
    cxx = (sxx - 2.0 * pxi * s1x + n * pxi * pxi) * nrcp
    cyy = (syy - 2.0 * pyi * s1y + n * pyi * pyi) * nrcp
    czz = (szz - 2.0 * pzi * s1z + n * pzi * pzi) * nrcp
    cxy = (sxy - pxi * s1y - pyi * s1x + n * pxi * pyi) * nrcp
    cxz = (sxz - pxi * s1z - pzi * s1x + n * pxi * pzi) * nrcp
    cyz = (syz - pyi * s1z - pzi * s1y + n * pyi * pzi) * nrcp

    valid = n >= 3.0
    one = jnp.ones_like(cxx)
    zero = jnp.zeros_like(cxx)
    cxx = jnp.where(valid, cxx, one)
    cyy = jnp.where(valid, cyy, one)
    czz = jnp.where(valid, czz, one)
    cxy = jnp.where(valid, cxy, zero)
    cxz = jnp.where(valid, cxz, zero)
    cyz = jnp.where(valid, cyz, zero)

    q = (cxx + cyy + czz) / 3.0
    p1 = cxy * cxy + cxz * cxz + cyz * cyz
    dx0 = cxx - q
    dy0 = cyy - q
    dz0 = czz - q
    p2 = dx0 * dx0 + dy0 * dy0 + dz0 * dz0 + 2.0 * p1
    pmag = _sqrt(p2 / 6.0)
    pinv = _rcp(jnp.maximum(pmag, 1e-30))
    bxx = dx0 * pinv
    byy = dy0 * pinv
    bzz = dz0 * pinv
    bxy = cxy * pinv
    bxz = cxz * pinv
    byz = cyz * pinv
    detb = (bxx * (byy * bzz - byz * byz)
            - bxy * (bxy * bzz - byz * bxz)
            + bxz * (bxy * byz - byy * bxz))
    r = jnp.clip(detb * 0.5, -1.0, 1.0)
    tmax = jnp.full_like(r, 2.0)
    tmin = jnp.full_like(r, -2.0)
    for _ in range(24):
        f = (tmax * tmax - 3.0) * tmax - 2.0 * r
        fp = 3.0 * tmax * tmax - 3.0
        tmax = tmax - f / jnp.maximum(fp, 1e-30)
        f = (tmin * tmin - 3.0) * tmin - 2.0 * r
        fp = 3.0 * tmin * tmin - 3.0
        tmin = tmin - f / jnp.maximum(fp, 1e-30)
    e1 = q + pmag * tmax
    e3 = q + pmag * tmin
    e2 = 3.0 * q - e1 - e3

    r0 = (cxx - e3, cxy, cxz)
    r1 = (cxy, cyy - e3, cyz)
    r2 = (cxz, cyz, czz - e3)
    v01 = _cross(r0, r1)
    v02 = _cross(r0, r2)
    v12 = _cross(r1, r2)
    n01 = v01[0] * v01[0] + v01[1] * v01[1] + v01[2] * v01[2]
    n02 = v02[0] * v02[0] + v02[1] * v02[1] + v02[2] * v02[2]
    n12 = v12[0] * v12[0] + v12[1] * v12[1] + v12[2] * v12[2]
    use02 = n02 > n01
    nbest = jnp.where(use02, n02, n01)
    vx = jnp.where(use02, v02[0], v01[0])
    vy = jnp.where(use02, v02[1], v01[1])
    vz = jnp.where(use02, v02[2], v01[2])
    use12 = n12 > nbest
    nbest = jnp.where(use12, n12, nbest)
    vx = jnp.where(use12, v12[0], vx)
    vy = jnp.where(use12, v12[1], vy)
    vz = jnp.where(use12, v12[2], vz)
    inv = _rsqrt(jnp.maximum(nbest, 1e-38))
    vx = vx * inv
    vy = vy * inv
    vz = vz * inv
    d = vx * pxi + vy * pyi + vz * pzi
    flip = jnp.where(d > 0.0, -1.0, 1.0)
    nvalid = valid
    nx = jnp.where(nvalid, vx * flip, 0.0)
    ny = jnp.where(nvalid, vy * flip, 0.0)
    nz = jnp.where(nvalid, vz * flip, 0.0)

    a1 = jnp.abs(e1)
    a2 = jnp.abs(e2)
    a3 = jnp.abs(e3)
    t0 = jnp.maximum(a1, a2)
    t1 = jnp.minimum(a1, a2)
    ee0 = jnp.maximum(t0, a3)
    tm = jnp.minimum(t0, a3)
    ee1 = jnp.maximum(t1, tm)
    ee2 = jnp.minimum(t1, tm)
    s = ee0 + ee1 + ee2
    sinv = _rcp(jnp.maximum(s, 1e-12))
    en0 = ee0 * sinv
    en1 = ee1 * sinv
    en2 = ee2 * sinv
    den = _rcp(en0 + 1e-6)
    cgate = nvalid & (s > 1e-6)
    lin = jnp.where(cgate, (en0 - en1) * den, 0.0)
    pla = jnp.where(cgate, (en1 - en2) * den, 0.0)
    sph = jnp.where(cgate, en2 * den, 0.0)

    h = jnp.concatenate([pxi, pyi, pzi, nx, ny, nz, lin, pla, sph], axis=0)

    a1s = g1[...] * _rsqrt(rv1[...] + _EPS)
    c1s = b1[...] - rm1[...] * a1s
    z = jax.lax.dot_general(w1[...], h, (((1,), (0,)), ((), ())),
                            preferred_element_type=jnp.float32,
                            precision=jax.lax.Precision.HIGHEST)
    z = jnp.maximum(z * a1s + c1s, 0.0)
    a2s = g2[...] * _rsqrt(rv2[...] + _EPS)
    c2s = b2[...] - rm2[...] * a2s
    z = jax.lax.dot_general(w2[...], z, (((1,), (0,)), ((), ())),
                            preferred_element_type=jnp.float32,
                            precision=jax.lax.Precision.HIGHEST)
    z = jnp.maximum(z * a2s + c2s, 0.0)
    z = jax.lax.dot_general(w3[...], z, (((1,), (0,)), ((), ())),
                            preferred_element_type=jnp.float32,
                            precision=jax.lax.Precision.HIGHEST)
    z = z + b3[...]
    out_ref[...] = z.T


def kernel(features, coors, W1, g1, b1, rm1, rv1, W2, g2, b2, rm2, rv2, W3, b3):
    feat_t = features.T
    coors_col = coors[:, :3].astype(jnp.int32)
    coors_row = coors_col.T

    full = lambda shape: pl.BlockSpec(shape, lambda i: (0, 0))
    out = pl.pallas_call(
        _body,
        grid=(_N // _BI,),
        in_specs=[
            full((3, _N)),
            full((_N, 3)),
            full((3, _N)),
            full((64, 9)),
            full((64, 1)), full((64, 1)), full((64, 1)), full((64, 1)),
            full((128, 64)),
            full((128, 1)), full((128, 1)), full((128, 1)), full((128, 1)),
            full((128, 128)),
            full((128, 1)),
        ],
        out_specs=pl.BlockSpec((_BI, 128), lambda i: (i, 0)),
        out_shape=jax.ShapeDtypeStruct((_N, 128), jnp.float32),
    )(feat_t, coors_col, coors_row, W1,
      g1.reshape(64, 1), b1.reshape(64, 1), rm1.reshape(64, 1), rv1.reshape(64, 1),
      W2,
      g2.reshape(128, 1), b2.reshape(128, 1), rm2.reshape(128, 1), rv2.reshape(128, 1),
      W3, b3.reshape(128, 1))
    return out


# speedup vs baseline: 165.1129x; 1.2321x over previous
import jax
import jax.numpy as jnp
from jax.experimental import pallas as pl
from jax.experimental.pallas import tpu as pltpu

_N = 8192
_BI = 512
_BJ = 2048
_R = 3
_EPS = 1e-5


def _cross(a, b):
    return (a[1] * b[2] - a[2] * b[1],
            a[2] * b[0] - a[0] * b[2],
            a[0] * b[1] - a[1] * b[0])


def _rcp(x):
    r = 1.0 / x
    r = r * (2.0 - x * r)
    r = r * (2.0 - x * r)
    return r


def _rsqrt(x):
    y = jax.lax.rsqrt(x)
    y = y * (1.5 - 0.5 * x * y * y)
    y = y * (1.5 - 0.5 * x * y * y)
    return y


def _sqrt(x):
    xs = jnp.maximum(x, 1e-38)
    return xs * _rsqrt(xs)


def _body(feat_t, coors_col, coors_row, w1, g1, b1, rm1, rv1,
          w2, g2, b2, rm2, rv2, w3, b3, out_ref):
    i = pl.program_id(0)
    i0 = i * _BI

    px_all = feat_t[0:1, :]
    py_all = feat_t[1:2, :]
    pz_all = feat_t[2:3, :]
    xt = jnp.concatenate([
        jnp.ones_like(px_all), px_all, py_all, pz_all,
        px_all * px_all, px_all * py_all, px_all * pz_all,
        py_all * py_all, py_all * pz_all, pz_all * pz_all,
    ], axis=0)
    xt0 = xt.astype(jnp.bfloat16)

    bi = coors_row[0:1, pl.ds(i0, _BI)]
    yi = coors_row[1:2, pl.ds(i0, _BI)]
    xi = coors_row[2:3, pl.ds(i0, _BI)]

    bj = coors_col[:, 0:1]
    yj = coors_col[:, 1:2]
    xj = coors_col[:, 2:3]
    rr = jnp.bfloat16(_R)
    m = ((bj == bi)
         & (jnp.abs(yj - yi) <= rr)
         & (jnp.abs(xj - xi) <= rr)).astype(jnp.bfloat16)
    acc = jax.lax.dot_general(
        xt0, m, (((1,), (0,)), ((), ())),
        preferred_element_type=jnp.float32)

    pxi = feat_t[0:1, pl.ds(i0, _BI)]
    pyi = feat_t[1:2, pl.ds(i0, _BI)]
    pzi = feat_t[2:3, pl.ds(i0, _BI)]
    _b = lambda v: v.astype(jnp.bfloat16).astype(jnp.float32)
    n = acc[0:1] - 1.0
    s1x = acc[1:2] - _b(pxi)
    s1y = acc[2:3] - _b(pyi)
    s1z = acc[3:4] - _b(pzi)
    sxx = acc[4:5] - _b(pxi * pxi)
    sxy = acc[5:6] - _b(pxi * pyi)
    sxz = acc[6:7] - _b(pxi * pzi)
    syy = acc[7:8] - _b(pyi * pyi)
    syz = acc[8:9] - _b(pyi * pzi)
    szz = acc[9:10] - _b(pzi * pzi)

    nrcp = _rcp(jnp.maximum(n, 1.0))
    cxx = (sxx - 2.0 * pxi * s1x + n * pxi * pxi) * nrcp
    cyy = (syy - 2.0 * pyi * s1y + n * pyi * pyi) * nrcp
    czz = (szz - 2.0 * pzi * s1z + n * pzi * pzi) * nrcp
    cxy = (sxy - pxi * s1y - pyi * s1x + n * pxi * pyi) * nrcp
    cxz = (sxz - pxi * s1z - pzi * s1x + n * pxi * pzi) * nrcp
    cyz = (syz - pyi * s1z - pzi * s1y + n * pyi * pzi) * nrcp

    valid = n >= 3.0
    one = jnp.ones_like(cxx)
    zero = jnp.zeros_like(cxx)
    cxx = jnp.where(valid, cxx, one)
    cyy = jnp.where(valid, cyy, one)
    czz = jnp.where(valid, czz, one)
    cxy = jnp.where(valid, cxy, zero)
    cxz = jnp.where(valid, cxz, zero)
    cyz = jnp.where(valid, cyz, zero)

    q = (cxx + cyy + czz) / 3.0
    p1 = cxy * cxy + cxz * cxz + cyz * cyz
    dx0 = cxx - q
    dy0 = cyy - q
    dz0 = czz - q
    p2 = dx0 * dx0 + dy0 * dy0 + dz0 * dz0 + 2.0 * p1
    pmag = _sqrt(p2 / 6.0)
    pinv = _rcp(jnp.maximum(pmag, 1e-30))
    bxx = dx0 * pinv
    byy = dy0 * pinv
    bzz = dz0 * pinv
    bxy = cxy * pinv
    bxz = cxz * pinv
    byz = cyz * pinv
    detb = (bxx * (byy * bzz - byz * byz)
            - bxy * (bxy * bzz - byz * bxz)
            + bxz * (bxy * byz - byy * bxz))
    r = jnp.clip(detb * 0.5, -1.0, 1.0)
    tmax = jnp.full_like(r, 2.0)
    tmin = jnp.full_like(r, -2.0)
    for _ in range(24):
        f = (tmax * tmax - 3.0) * tmax - 2.0 * r
        fp = 3.0 * tmax * tmax - 3.0
        tmax = tmax - f / jnp.maximum(fp, 1e-30)
        f = (tmin * tmin - 3.0) * tmin - 2.0 * r
        fp = 3.0 * tmin * tmin - 3.0
        tmin = tmin - f / jnp.maximum(fp, 1e-30)
    e1 = q + pmag * tmax
    e3 = q + pmag * tmin
    e2 = 3.0 * q - e1 - e3

    r0 = (cxx - e3, cxy, cxz)
    r1 = (cxy, cyy - e3, cyz)
    r2 = (cxz, cyz, czz - e3)
    v01 = _cross(r0, r1)
    v02 = _cross(r0, r2)
    v12 = _cross(r1, r2)
    n01 = v01[0] * v01[0] + v01[1] * v01[1] + v01[2] * v01[2]
    n02 = v02[0] * v02[0] + v02[1] * v02[1] + v02[2] * v02[2]
    n12 = v12[0] * v12[0] + v12[1] * v12[1] + v12[2] * v12[2]
    use02 = n02 > n01
    nbest = jnp.where(use02, n02, n01)
    vx = jnp.where(use02, v02[0], v01[0])
    vy = jnp.where(use02, v02[1], v01[1])
    vz = jnp.where(use02, v02[2], v01[2])
    use12 = n12 > nbest
    nbest = jnp.where(use12, n12, nbest)
    vx = jnp.where(use12, v12[0], vx)
    vy = jnp.where(use12, v12[1], vy)
    vz = jnp.where(use12, v12[2], vz)
    inv = _rsqrt(jnp.maximum(nbest, 1e-38))
    vx = vx * inv
    vy = vy * inv
    vz = vz * inv
    d = vx * pxi + vy * pyi + vz * pzi
    flip = jnp.where(d > 0.0, -1.0, 1.0)
    nvalid = valid
    nx = jnp.where(nvalid, vx * flip, 0.0)
    ny = jnp.where(nvalid, vy * flip, 0.0)
    nz = jnp.where(nvalid, vz * flip, 0.0)

    a1 = jnp.abs(e1)
    a2 = jnp.abs(e2)
    a3 = jnp.abs(e3)
    t0 = jnp.maximum(a1, a2)
    t1 = jnp.minimum(a1, a2)
    ee0 = jnp.maximum(t0, a3)
    tm = jnp.minimum(t0, a3)
    ee1 = jnp.maximum(t1, tm)
    ee2 = jnp.minimum(t1, tm)
    s = ee0 + ee1 + ee2
    sinv = _rcp(jnp.maximum(s, 1e-12))
    en0 = ee0 * sinv
    en1 = ee1 * sinv
    en2 = ee2 * sinv
    den = _rcp(en0 + 1e-6)
    cgate = nvalid & (s > 1e-6)
    lin = jnp.where(cgate, (en0 - en1) * den, 0.0)
    pla = jnp.where(cgate, (en1 - en2) * den, 0.0)
    sph = jnp.where(cgate, en2 * den, 0.0)

    h = jnp.concatenate([pxi, pyi, pzi, nx, ny, nz, lin, pla, sph], axis=0)

    a1s = g1[...] * _rsqrt(rv1[...] + _EPS)
    c1s = b1[...] - rm1[...] * a1s
    z = jax.lax.dot_general(w1[...], h, (((1,), (0,)), ((), ())),
                            preferred_element_type=jnp.float32,
                            precision=jax.lax.Precision.HIGHEST)
    z = jnp.maximum(z * a1s + c1s, 0.0)
    a2s = g2[...] * _rsqrt(rv2[...] + _EPS)
    c2s = b2[...] - rm2[...] * a2s
    z = jax.lax.dot_general(w2[...], z, (((1,), (0,)), ((), ())),
                            preferred_element_type=jnp.float32,
                            precision=jax.lax.Precision.HIGHEST)
    z = jnp.maximum(z * a2s + c2s, 0.0)
    z = jax.lax.dot_general(w3[...], z, (((1,), (0,)), ((), ())),
                            preferred_element_type=jnp.float32,
                            precision=jax.lax.Precision.HIGHEST)
    z = z + b3[...]
    out_ref[...] = z.T


def kernel(features, coors, W1, g1, b1, rm1, rv1, W2, g2, b2, rm2, rv2, W3, b3):
    feat_t = features.T
    coors_col = coors[:, :3].astype(jnp.bfloat16)
    coors_row = coors_col.T

    full = lambda shape: pl.BlockSpec(shape, lambda i: (0, 0))
    out = pl.pallas_call(
        _body,
        grid=(_N // _BI,),
        in_specs=[
            full((3, _N)),
            full((_N, 3)),
            full((3, _N)),
            full((64, 9)),
            full((64, 1)), full((64, 1)), full((64, 1)), full((64, 1)),
            full((128, 64)),
            full((128, 1)), full((128, 1)), full((128, 1)), full((128, 1)),
            full((128, 128)),
            full((128, 1)),
        ],
        out_specs=pl.BlockSpec((_BI, 128), lambda i: (i, 0)),
        out_shape=jax.ShapeDtypeStruct((_N, 128), jnp.float32),
    )(feat_t, coors_col, coors_row, W1,
      g1.reshape(64, 1), b1.reshape(64, 1), rm1.reshape(64, 1), rv1.reshape(64, 1),
      W2,
      g2.reshape(128, 1), b2.reshape(128, 1), rm2.reshape(128, 1), rv2.reshape(128, 1),
      W3, b3.reshape(128, 1))
    return out


# speedup vs baseline: 171.3818x; 1.0380x over previous
import jax
import jax.numpy as jnp
from jax.experimental import pallas as pl

_N = 8192
_BI = 512
_R = 3
_EPS = 1e-5


def _cross(a, b):
    return (a[1] * b[2] - a[2] * b[1],
            a[2] * b[0] - a[0] * b[2],
            a[0] * b[1] - a[1] * b[0])


def _rcp(x):
    r = 1.0 / x
    r = r * (2.0 - x * r)
    r = r * (2.0 - x * r)
    return r


def _rsqrt(x):
    y = jax.lax.rsqrt(x)
    y = y * (1.5 - 0.5 * x * y * y)
    y = y * (1.5 - 0.5 * x * y * y)
    return y


def _sqrt(x):
    xs = jnp.maximum(x, 1e-38)
    return xs * _rsqrt(xs)


def _body(feat_t, coors_col, coors_row, w1, g1, b1, rm1, rv1,
          w2, g2, b2, rm2, rv2, w3, b3, out_ref):
    i = pl.program_id(0)
    i0 = i * _BI

    px_all = feat_t[0:1, :]
    py_all = feat_t[1:2, :]
    pz_all = feat_t[2:3, :]
    xt = jnp.concatenate([
        jnp.ones_like(px_all), px_all, py_all, pz_all,
        px_all * px_all, px_all * py_all, px_all * pz_all,
        py_all * py_all, py_all * pz_all, pz_all * pz_all,
    ], axis=0)
    xt0 = xt.astype(jnp.bfloat16)

    bi = coors_row[0:1, pl.ds(i0, _BI)]
    yi = coors_row[1:2, pl.ds(i0, _BI)]
    xi = coors_row[2:3, pl.ds(i0, _BI)]

    bj = coors_col[:, 0:1]
    yj = coors_col[:, 1:2]
    xj = coors_col[:, 2:3]
    zero_b = jnp.bfloat16(0.0)
    one_b = jnp.bfloat16(1.0)
    four_b = jnp.bfloat16(4.0)
    ty = jnp.maximum(four_b - jnp.abs(yj - yi), zero_b)
    tx = jnp.maximum(four_b - jnp.abs(xj - xi), zero_b)
    tb = jnp.maximum(one_b - jnp.abs(bj - bi), zero_b)
    m = jnp.minimum(ty * tx, one_b) * tb
    acc = jax.lax.dot_general(
        xt0, m, (((1,), (0,)), ((), ())),
        preferred_element_type=jnp.float32)

    pxi = feat_t[0:1, pl.ds(i0, _BI)]
    pyi = feat_t[1:2, pl.ds(i0, _BI)]
    pzi = feat_t[2:3, pl.ds(i0, _BI)]
    _b = lambda v: v.astype(jnp.bfloat16).astype(jnp.float32)
    n = acc[0:1] - 1.0
    s1x = acc[1:2] - _b(pxi)
    s1y = acc[2:3] - _b(pyi)
    s1z = acc[3:4] - _b(pzi)
    sxx = acc[4:5] - _b(pxi * pxi)
    sxy = acc[5:6] - _b(pxi * pyi)
    sxz = acc[6:7] - _b(pxi * pzi)
    syy = acc[7:8] - _b(pyi * pyi)
    syz = acc[8:9] - _b(pyi * pzi)
    szz = acc[9:10] - _b(pzi * pzi)

    nrcp = _rcp(jnp.maximum(n, 1.0))
    cxx = (sxx - 2.0 * pxi * s1x + n * pxi * pxi) * nrcp
    cyy = (syy - 2.0 * pyi * s1y + n * pyi * pyi) * nrcp
    czz = (szz - 2.0 * pzi * s1z + n * pzi * pzi) * nrcp
    cxy = (sxy - pxi * s1y - pyi * s1x + n * pxi * pyi) * nrcp
    cxz = (sxz - pxi * s1z - pzi * s1x + n * pxi * pzi) * nrcp
    cyz = (syz - pyi * s1z - pzi * s1y + n * pyi * pzi) * nrcp

    valid = n >= 3.0
    one = jnp.ones_like(cxx)
    zero = jnp.zeros_like(cxx)
    cxx = jnp.where(valid, cxx, one)
    cyy = jnp.where(valid, cyy, one)
    czz = jnp.where(valid, czz, one)
    cxy = jnp.where(valid, cxy, zero)
    cxz = jnp.where(valid, cxz, zero)
    cyz = jnp.where(valid, cyz, zero)

    q = (cxx + cyy + czz) / 3.0
    p1 = cxy * cxy + cxz * cxz + cyz * cyz
    dx0 = cxx - q
    dy0 = cyy - q
    dz0 = czz - q
    p2 = dx0 * dx0 + dy0 * dy0 + dz0 * dz0 + 2.0 * p1
    pmag = _sqrt(p2 / 6.0)
    pinv = _rcp(jnp.maximum(pmag, 1e-30))
    bxx = dx0 * pinv
    byy = dy0 * pinv
    bzz = dz0 * pinv
    bxy = cxy * pinv
    bxz = cxz * pinv
    byz = cyz * pinv
    detb = (bxx * (byy * bzz - byz * byz)
            - bxy * (bxy * bzz - byz * bxz)
            + bxz * (bxy * byz - byy * bxz))
    r = jnp.clip(detb * 0.5, -1.0, 1.0)
    tmax = jnp.full_like(r, 2.0)
    tmin = jnp.full_like(r, -2.0)
    for _ in range(24):
        f = (tmax * tmax - 3.0) * tmax - 2.0 * r
        fp = 3.0 * tmax * tmax - 3.0
        tmax = tmax - f / jnp.maximum(fp, 1e-30)
        f = (tmin * tmin - 3.0) * tmin - 2.0 * r
        fp = 3.0 * tmin * tmin - 3.0
        tmin = tmin - f / jnp.maximum(fp, 1e-30)
    e1 = q + pmag * tmax
    e3 = q + pmag * tmin
    e2 = 3.0 * q - e1 - e3

    r0 = (cxx - e3, cxy, cxz)
    r1 = (cxy, cyy - e3, cyz)
    r2 = (cxz, cyz, czz - e3)
    v01 = _cross(r0, r1)
    v02 = _cross(r0, r2)
    v12 = _cross(r1, r2)
    n01 = v01[0] * v01[0] + v01[1] * v01[1] + v01[2] * v01[2]
    n02 = v02[0] * v02[0] + v02[1] * v02[1] + v02[2] * v02[2]
    n12 = v12[0] * v12[0] + v12[1] * v12[1] + v12[2] * v12[2]
    use02 = n02 > n01
    nbest = jnp.where(use02, n02, n01)
    vx = jnp.where(use02, v02[0], v01[0])
    vy = jnp.where(use02, v02[1], v01[1])
    vz = jnp.where(use02, v02[2], v01[2])
    use12 = n12 > nbest
    nbest = jnp.where(use12, n12, nbest)
    vx = jnp.where(use12, v12[0], vx)
    vy = jnp.where(use12, v12[1], vy)
    vz = jnp.where(use12, v12[2], vz)
    inv = _rsqrt(jnp.maximum(nbest, 1e-38))
    vx = vx * inv
    vy = vy * inv
    vz = vz * inv
    d = vx * pxi + vy * pyi + vz * pzi
    flip = jnp.where(d > 0.0, -1.0, 1.0)
    nvalid = valid
    nx = jnp.where(nvalid, vx * flip, 0.0)
    ny = jnp.where(nvalid, vy * flip, 0.0)
    nz = jnp.where(nvalid, vz * flip, 0.0)

    a1 = jnp.abs(e1)
    a2 = jnp.abs(e2)
    a3 = jnp.abs(e3)
    t0 = jnp.maximum(a1, a2)
    t1 = jnp.minimum(a1, a2)
    ee0 = jnp.maximum(t0, a3)
    tm = jnp.minimum(t0, a3)
    ee1 = jnp.maximum(t1, tm)
    ee2 = jnp.minimum(t1, tm)
    s = ee0 + ee1 + ee2
    sinv = _rcp(jnp.maximum(s, 1e-12))
    en0 = ee0 * sinv
    en1 = ee1 * sinv
    en2 = ee2 * sinv
    den = _rcp(en0 + 1e-6)
    cgate = nvalid & (s > 1e-6)
    lin = jnp.where(cgate, (en0 - en1) * den, 0.0)
    pla = jnp.where(cgate, (en1 - en2) * den, 0.0)
    sph = jnp.where(cgate, en2 * den, 0.0)

    h = jnp.concatenate([pxi, pyi, pzi, nx, ny, nz, lin, pla, sph], axis=0)

    a1s = g1[...] * _rsqrt(rv1[...] + _EPS)
    c1s = b1[...] - rm1[...] * a1s
    z = jax.lax.dot_general(w1[...], h, (((1,), (0,)), ((), ())),
                            preferred_element_type=jnp.float32,
                            precision=jax.lax.Precision.HIGHEST)
    z = jnp.maximum(z * a1s + c1s, 0.0)
    a2s = g2[...] * _rsqrt(rv2[...] + _EPS)
    c2s = b2[...] - rm2[...] * a2s
    z = jax.lax.dot_general(w2[...], z, (((1,), (0,)), ((), ())),
                            preferred_element_type=jnp.float32,
                            precision=jax.lax.Precision.HIGHEST)
    z = jnp.maximum(z * a2s + c2s, 0.0)
    z = jax.lax.dot_general(w3[...], z, (((1,), (0,)), ((), ())),
                            preferred_element_type=jnp.float32,
                            precision=jax.lax.Precision.HIGHEST)
    z = z + b3[...]
    out_ref[...] = z.T


def kernel(features, coors, W1, g1, b1, rm1, rv1, W2, g2, b2, rm2, rv2, W3, b3):
    feat_t = features.T
    coors_col = coors[:, :3].astype(jnp.bfloat16)
    coors_row = coors_col.T

    full = lambda shape: pl.BlockSpec(shape, lambda i: (0, 0))
    out = pl.pallas_call(
        _body,
        grid=(_N // _BI,),
        in_specs=[
            full((3, _N)),
            full((_N, 3)),
            full((3, _N)),
            full((64, 9)),
            full((64, 1)), full((64, 1)), full((64, 1)), full((64, 1)),
            full((128, 64)),
            full((128, 1)), full((128, 1)), full((128, 1)), full((128, 1)),
            full((128, 128)),
            full((128, 1)),
        ],
        out_specs=pl.BlockSpec((_BI, 128), lambda i: (i, 0)),
        out_shape=jax.ShapeDtypeStruct((_N, 128), jnp.float32),
    )(feat_t, coors_col, coors_row, W1,
      g1.reshape(64, 1), b1.reshape(64, 1), rm1.reshape(64, 1), rv1.reshape(64, 1),
      W2,
      g2.reshape(128, 1), b2.reshape(128, 1), rm2.reshape(128, 1), rv2.reshape(128, 1),
      W3, b3.reshape(128, 1))
    return out


# speedup vs baseline: 176.8845x; 1.0321x over previous
import jax
import jax.numpy as jnp
from jax.experimental import pallas as pl

_N = 8192
_BI = 1024
_R = 3
_EPS = 1e-5


def _cross(a, b):
    return (a[1] * b[2] - a[2] * b[1],
            a[2] * b[0] - a[0] * b[2],
            a[0] * b[1] - a[1] * b[0])


def _rcp(x):
    r = 1.0 / x
    r = r * (2.0 - x * r)
    r = r * (2.0 - x * r)
    return r


def _rsqrt(x):
    y = jax.lax.rsqrt(x)
    y = y * (1.5 - 0.5 * x * y * y)
    y = y * (1.5 - 0.5 * x * y * y)
    return y


def _sqrt(x):
    xs = jnp.maximum(x, 1e-38)
    return xs * _rsqrt(xs)


def _body(feat_t, coors_col, coors_row, w1, g1, b1, rm1, rv1,
          w2, g2, b2, rm2, rv2, w3, b3, out_ref):
    i = pl.program_id(0)
    i0 = i * _BI

    px_all = feat_t[0:1, :]
    py_all = feat_t[1:2, :]
    pz_all = feat_t[2:3, :]
    xt = jnp.concatenate([
        jnp.ones_like(px_all), px_all, py_all, pz_all,
        px_all * px_all, px_all * py_all, px_all * pz_all,
        py_all * py_all, py_all * pz_all, pz_all * pz_all,
    ], axis=0)
    xt0 = xt.astype(jnp.bfloat16)

    bi = coors_row[0:1, pl.ds(i0, _BI)]
    yi = coors_row[1:2, pl.ds(i0, _BI)]
    xi = coors_row[2:3, pl.ds(i0, _BI)]

    bj = coors_col[:, 0:1]
    yj = coors_col[:, 1:2]
    xj = coors_col[:, 2:3]
    zero_b = jnp.bfloat16(0.0)
    one_b = jnp.bfloat16(1.0)
    four_b = jnp.bfloat16(4.0)
    ty = jnp.maximum(four_b - jnp.abs(yj - yi), zero_b)
    tx = jnp.maximum(four_b - jnp.abs(xj - xi), zero_b)
    tb = jnp.maximum(one_b - jnp.abs(bj - bi), zero_b)
    m = jnp.minimum(ty * tx, one_b) * tb
    acc = jax.lax.dot_general(
        xt0, m, (((1,), (0,)), ((), ())),
        preferred_element_type=jnp.float32)

    pxi = feat_t[0:1, pl.ds(i0, _BI)]
    pyi = feat_t[1:2, pl.ds(i0, _BI)]
    pzi = feat_t[2:3, pl.ds(i0, _BI)]
    _b = lambda v: v.astype(jnp.bfloat16).astype(jnp.float32)
    n = acc[0:1] - 1.0
    s1x = acc[1:2] - _b(pxi)
    s1y = acc[2:3] - _b(pyi)
    s1z = acc[3:4] - _b(pzi)
    sxx = acc[4:5] - _b(pxi * pxi)
    sxy = acc[5:6] - _b(pxi * pyi)
    sxz = acc[6:7] - _b(pxi * pzi)
    syy = acc[7:8] - _b(pyi * pyi)
    syz = acc[8:9] - _b(pyi * pzi)
    szz = acc[9:10] - _b(pzi * pzi)

    nrcp = _rcp(jnp.maximum(n, 1.0))
    cxx = (sxx - 2.0 * pxi * s1x + n * pxi * pxi) * nrcp
    cyy = (syy - 2.0 * pyi * s1y + n * pyi * pyi) * nrcp
    czz = (szz - 2.0 * pzi * s1z + n * pzi * pzi) * nrcp
    cxy = (sxy - pxi * s1y - pyi * s1x + n * pxi * pyi) * nrcp
    cxz = (sxz - pxi * s1z - pzi * s1x + n * pxi * pzi) * nrcp
    cyz = (syz - pyi * s1z - pzi * s1y + n * pyi * pzi) * nrcp

    valid = n >= 3.0
    one = jnp.ones_like(cxx)
    zero = jnp.zeros_like(cxx)
    cxx = jnp.where(valid, cxx, one)
    cyy = jnp.where(valid, cyy, one)
    czz = jnp.where(valid, czz, one)
    cxy = jnp.where(valid, cxy, zero)
    cxz = jnp.where(valid, cxz, zero)
    cyz = jnp.where(valid, cyz, zero)

    q = (cxx + cyy + czz) / 3.0
    p1 = cxy * cxy + cxz * cxz + cyz * cyz
    dx0 = cxx - q
    dy0 = cyy - q
    dz0 = czz - q
    p2 = dx0 * dx0 + dy0 * dy0 + dz0 * dz0 + 2.0 * p1
    pmag = _sqrt(p2 / 6.0)
    pinv = _rcp(jnp.maximum(pmag, 1e-30))
    bxx = dx0 * pinv
    byy = dy0 * pinv
    bzz = dz0 * pinv
    bxy = cxy * pinv
    bxz = cxz * pinv
    byz = cyz * pinv
    detb = (bxx * (byy * bzz - byz * byz)
            - bxy * (bxy * bzz - byz * bxz)
            + bxz * (bxy * byz - byy * bxz))
    r = jnp.clip(detb * 0.5, -1.0, 1.0)
    tmax = jnp.full_like(r, 2.0)
    tmin = jnp.full_like(r, -2.0)
    for _ in range(24):
        f = (tmax * tmax - 3.0) * tmax - 2.0 * r
        fp = 3.0 * tmax * tmax - 3.0
        tmax = tmax - f / jnp.maximum(fp, 1e-30)
        f = (tmin * tmin - 3.0) * tmin - 2.0 * r
        fp = 3.0 * tmin * tmin - 3.0
        tmin = tmin - f / jnp.maximum(fp, 1e-30)
    e1 = q + pmag * tmax
    e3 = q + pmag * tmin
    e2 = 3.0 * q - e1 - e3

    r0 = (cxx - e3, cxy, cxz)
    r1 = (cxy, cyy - e3, cyz)
    r2 = (cxz, cyz, czz - e3)
    v01 = _cross(r0, r1)
    v02 = _cross(r0, r2)
    v12 = _cross(r1, r2)
    n01 = v01[0] * v01[0] + v01[1] * v01[1] + v01[2] * v01[2]
    n02 = v02[0] * v02[0] + v02[1] * v02[1] + v02[2] * v02[2]
    n12 = v12[0] * v12[0] + v12[1] * v12[1] + v12[2] * v12[2]
    use02 = n02 > n01
    nbest = jnp.where(use02, n02, n01)
    vx = jnp.where(use02, v02[0], v01[0])
    vy = jnp.where(use02, v02[1], v01[1])
    vz = jnp.where(use02, v02[2], v01[2])
    use12 = n12 > nbest
    nbest = jnp.where(use12, n12, nbest)
    vx = jnp.where(use12, v12[0], vx)
    vy = jnp.where(use12, v12[1], vy)
    vz = jnp.where(use12, v12[2], vz)
    inv = _rsqrt(jnp.maximum(nbest, 1e-38))
    vx = vx * inv
    vy = vy * inv
    vz = vz * inv
    d = vx * pxi + vy * pyi + vz * pzi
    flip = jnp.where(d > 0.0, -1.0, 1.0)
    nvalid = valid
    nx = jnp.where(nvalid, vx * flip, 0.0)
    ny = jnp.where(nvalid, vy * flip, 0.0)
    nz = jnp.where(nvalid, vz * flip, 0.0)

    a1 = jnp.abs(e1)
    a2 = jnp.abs(e2)
    a3 = jnp.abs(e3)
    t0 = jnp.maximum(a1, a2)
    t1 = jnp.minimum(a1, a2)
    ee0 = jnp.maximum(t0, a3)
    tm = jnp.minimum(t0, a3)
    ee1 = jnp.maximum(t1, tm)
    ee2 = jnp.minimum(t1, tm)
    s = ee0 + ee1 + ee2
    sinv = _rcp(jnp.maximum(s, 1e-12))
    en0 = ee0 * sinv
    en1 = ee1 * sinv
    en2 = ee2 * sinv
    den = _rcp(en0 + 1e-6)
    cgate = nvalid & (s > 1e-6)
    lin = jnp.where(cgate, (en0 - en1) * den, 0.0)
    pla = jnp.where(cgate, (en1 - en2) * den, 0.0)
    sph = jnp.where(cgate, en2 * den, 0.0)

    h = jnp.concatenate([pxi, pyi, pzi, nx, ny, nz, lin, pla, sph], axis=0)

    a1s = g1[...] * _rsqrt(rv1[...] + _EPS)
    c1s = b1[...] - rm1[...] * a1s
    z = jax.lax.dot_general(w1[...], h, (((1,), (0,)), ((), ())),
                            preferred_element_type=jnp.float32,
                            precision=jax.lax.Precision.HIGHEST)
    z = jnp.maximum(z * a1s + c1s, 0.0)
    a2s = g2[...] * _rsqrt(rv2[...] + _EPS)
    c2s = b2[...] - rm2[...] * a2s
    z = jax.lax.dot_general(w2[...], z, (((1,), (0,)), ((), ())),
                            preferred_element_type=jnp.float32,
                            precision=jax.lax.Precision.HIGHEST)
    z = jnp.maximum(z * a2s + c2s, 0.0)
    z = jax.lax.dot_general(w3[...], z, (((1,), (0,)), ((), ())),
                            preferred_element_type=jnp.float32,
                            precision=jax.lax.Precision.HIGHEST)
    z = z + b3[...]
    out_ref[...] = z.T


def kernel(features, coors, W1, g1, b1, rm1, rv1, W2, g2, b2, rm2, rv2, W3, b3):
    feat_t = features.T
    coors_col = coors[:, :3].astype(jnp.bfloat16)
    coors_row = coors_col.T

    full = lambda shape: pl.BlockSpec(shape, lambda i: (0, 0))
    out = pl.pallas_call(
        _body,
        grid=(_N // _BI,),
        in_specs=[
            full((3, _N)),
            full((_N, 3)),
            full((3, _N)),
            full((64, 9)),
            full((64, 1)), full((64, 1)), full((64, 1)), full((64, 1)),
            full((128, 64)),
            full((128, 1)), full((128, 1)), full((128, 1)), full((128, 1)),
            full((128, 128)),
            full((128, 1)),
        ],
        out_specs=pl.BlockSpec((_BI, 128), lambda i: (i, 0)),
        out_shape=jax.ShapeDtypeStruct((_N, 128), jnp.float32),
    )(feat_t, coors_col, coors_row, W1,
      g1.reshape(64, 1), b1.reshape(64, 1), rm1.reshape(64, 1), rv1.reshape(64, 1),
      W2,
      g2.reshape(128, 1), b2.reshape(128, 1), rm2.reshape(128, 1), rv2.reshape(128, 1),
      W3, b3.reshape(128, 1))
    return out


# speedup vs baseline: 186.1878x; 1.0526x over previous
import jax
import jax.numpy as jnp
from jax.experimental import pallas as pl

_N = 8192
_BI = 2048
_R = 3
_EPS = 1e-5


def _cross(a, b):
    return (a[1] * b[2] - a[2] * b[1],
            a[2] * b[0] - a[0] * b[2],
            a[0] * b[1] - a[1] * b[0])


def _rcp(x):
    r = 1.0 / x
    r = r * (2.0 - x * r)
    r = r * (2.0 - x * r)
    return r


def _rsqrt(x):
    y = jax.lax.rsqrt(x)
    y = y * (1.5 - 0.5 * x * y * y)
    y = y * (1.5 - 0.5 * x * y * y)
    return y


def _sqrt(x):
    xs = jnp.maximum(x, 1e-38)
    return xs * _rsqrt(xs)


def _body(feat_t, coors_col, coors_row, w1, g1, b1, rm1, rv1,
          w2, g2, b2, rm2, rv2, w3, b3, out_ref):
    i = pl.program_id(0)
    i0 = i * _BI

    px_all = feat_t[0:1, :]
    py_all = feat_t[1:2, :]
    pz_all = feat_t[2:3, :]
    xt = jnp.concatenate([
        jnp.ones_like(px_all), px_all, py_all, pz_all,
        px_all * px_all, px_all * py_all, px_all * pz_all,
        py_all * py_all, py_all * pz_all, pz_all * pz_all,
    ], axis=0)
    xt0 = xt.astype(jnp.bfloat16)

    bi = coors_row[0:1, pl.ds(i0, _BI)]
    yi = coors_row[1:2, pl.ds(i0, _BI)]
    xi = coors_row[2:3, pl.ds(i0, _BI)]

    bj = coors_col[:, 0:1]
    yj = coors_col[:, 1:2]
    xj = coors_col[:, 2:3]
    zero_b = jnp.bfloat16(0.0)
    one_b = jnp.bfloat16(1.0)
    four_b = jnp.bfloat16(4.0)
    ty = jnp.maximum(four_b - jnp.abs(yj - yi), zero_b)
    tx = jnp.maximum(four_b - jnp.abs(xj - xi), zero_b)
    tb = jnp.maximum(one_b - jnp.abs(bj - bi), zero_b)
    m = jnp.minimum(ty * tx, one_b) * tb
    acc = jax.lax.dot_general(
        xt0, m, (((1,), (0,)), ((), ())),
        preferred_element_type=jnp.float32)

    pxi = feat_t[0:1, pl.ds(i0, _BI)]
    pyi = feat_t[1:2, pl.ds(i0, _BI)]
    pzi = feat_t[2:3, pl.ds(i0, _BI)]
    _b = lambda v: v.astype(jnp.bfloat16).astype(jnp.float32)
    n = acc[0:1] - 1.0
    s1x = acc[1:2] - _b(pxi)
    s1y = acc[2:3] - _b(pyi)
    s1z = acc[3:4] - _b(pzi)
    sxx = acc[4:5] - _b(pxi * pxi)
    sxy = acc[5:6] - _b(pxi * pyi)
    sxz = acc[6:7] - _b(pxi * pzi)
    syy = acc[7:8] - _b(pyi * pyi)
    syz = acc[8:9] - _b(pyi * pzi)
    szz = acc[9:10] - _b(pzi * pzi)

    nrcp = _rcp(jnp.maximum(n, 1.0))
    cxx = (sxx - 2.0 * pxi * s1x + n * pxi * pxi) * nrcp
    cyy = (syy - 2.0 * pyi * s1y + n * pyi * pyi) * nrcp
    czz = (szz - 2.0 * pzi * s1z + n * pzi * pzi) * nrcp
    cxy = (sxy - pxi * s1y - pyi * s1x + n * pxi * pyi) * nrcp
    cxz = (sxz - pxi * s1z - pzi * s1x + n * pxi * pzi) * nrcp
    cyz = (syz - pyi * s1z - pzi * s1y + n * pyi * pzi) * nrcp

    valid = n >= 3.0
    one = jnp.ones_like(cxx)
    zero = jnp.zeros_like(cxx)
    cxx = jnp.where(valid, cxx, one)
    cyy = jnp.where(valid, cyy, one)
    czz = jnp.where(valid, czz, one)
    cxy = jnp.where(valid, cxy, zero)
    cxz = jnp.where(valid, cxz, zero)
    cyz = jnp.where(valid, cyz, zero)

    q = (cxx + cyy + czz) / 3.0
    p1 = cxy * cxy + cxz * cxz + cyz * cyz
    dx0 = cxx - q
    dy0 = cyy - q
    dz0 = czz - q
    p2 = dx0 * dx0 + dy0 * dy0 + dz0 * dz0 + 2.0 * p1
    pmag = _sqrt(p2 / 6.0)
    pinv = _rcp(jnp.maximum(pmag, 1e-30))
    bxx = dx0 * pinv
    byy = dy0 * pinv
    bzz = dz0 * pinv
    bxy = cxy * pinv
    bxz = cxz * pinv
    byz = cyz * pinv
    detb = (bxx * (byy * bzz - byz * byz)
            - bxy * (bxy * bzz - byz * bxz)
            + bxz * (bxy * byz - byy * bxz))
    r = jnp.clip(detb * 0.5, -1.0, 1.0)
    tmax = jnp.full_like(r, 2.0)
    tmin = jnp.full_like(r, -2.0)
    for _ in range(24):
        f = (tmax * tmax - 3.0) * tmax - 2.0 * r
        fp = 3.0 * tmax * tmax - 3.0
        tmax = tmax - f / jnp.maximum(fp, 1e-30)
        f = (tmin * tmin - 3.0) * tmin - 2.0 * r
        fp = 3.0 * tmin * tmin - 3.0
        tmin = tmin - f / jnp.maximum(fp, 1e-30)
    e1 = q + pmag * tmax
    e3 = q + pmag * tmin
    e2 = 3.0 * q - e1 - e3

    r0 = (cxx - e3, cxy, cxz)
    r1 = (cxy, cyy - e3, cyz)
    r2 = (cxz, cyz, czz - e3)
    v01 = _cross(r0, r1)
    v02 = _cross(r0, r2)
    v12 = _cross(r1, r2)
    n01 = v01[0] * v01[0] + v01[1] * v01[1] + v01[2] * v01[2]
    n02 = v02[0] * v02[0] + v02[1] * v02[1] + v02[2] * v02[2]
    n12 = v12[0] * v12[0] + v12[1] * v12[1] + v12[2] * v12[2]
    use02 = n02 > n01
    nbest = jnp.where(use02, n02, n01)
    vx = jnp.where(use02, v02[0], v01[0])
    vy = jnp.where(use02, v02[1], v01[1])
    vz = jnp.where(use02, v02[2], v01[2])
    use12 = n12 > nbest
    nbest = jnp.where(use12, n12, nbest)
    vx = jnp.where(use12, v12[0], vx)
    vy = jnp.where(use12, v12[1], vy)
    vz = jnp.where(use12, v12[2], vz)
    inv = _rsqrt(jnp.maximum(nbest, 1e-38))
    vx = vx * inv
    vy = vy * inv
    vz = vz * inv
    d = vx * pxi + vy * pyi + vz * pzi
    flip = jnp.where(d > 0.0, -1.0, 1.0)
    nvalid = valid
    nx = jnp.where(nvalid, vx * flip, 0.0)
    ny = jnp.where(nvalid, vy * flip, 0.0)
    nz = jnp.where(nvalid, vz * flip, 0.0)

    a1 = jnp.abs(e1)
    a2 = jnp.abs(e2)
    a3 = jnp.abs(e3)
    t0 = jnp.maximum(a1, a2)
    t1 = jnp.minimum(a1, a2)
    ee0 = jnp.maximum(t0, a3)
    tm = jnp.minimum(t0, a3)
    ee1 = jnp.maximum(t1, tm)
    ee2 = jnp.minimum(t1, tm)
    s = ee0 + ee1 + ee2
    sinv = _rcp(jnp.maximum(s, 1e-12))
    en0 = ee0 * sinv
    en1 = ee1 * sinv
    en2 = ee2 * sinv
    den = _rcp(en0 + 1e-6)
    cgate = nvalid & (s > 1e-6)
    lin = jnp.where(cgate, (en0 - en1) * den, 0.0)
    pla = jnp.where(cgate, (en1 - en2) * den, 0.0)
    sph = jnp.where(cgate, en2 * den, 0.0)

    h = jnp.concatenate([pxi, pyi, pzi, nx, ny, nz, lin, pla, sph], axis=0)

    a1s = g1[...] * _rsqrt(rv1[...] + _EPS)
    c1s = b1[...] - rm1[...] * a1s
    z = jax.lax.dot_general(w1[...], h, (((1,), (0,)), ((), ())),
                            preferred_element_type=jnp.float32,
                            precision=jax.lax.Precision.HIGHEST)
    z = jnp.maximum(z * a1s + c1s, 0.0)
    a2s = g2[...] * _rsqrt(rv2[...] + _EPS)
    c2s = b2[...] - rm2[...] * a2s
    z = jax.lax.dot_general(w2[...], z, (((1,), (0,)), ((), ())),
                            preferred_element_type=jnp.float32,
                            precision=jax.lax.Precision.HIGHEST)
    z = jnp.maximum(z * a2s + c2s, 0.0)
    z = jax.lax.dot_general(w3[...], z, (((1,), (0,)), ((), ())),
                            preferred_element_type=jnp.float32,
                            precision=jax.lax.Precision.HIGHEST)
    z = z + b3[...]
    out_ref[...] = z.T


def kernel(features, coors, W1, g1, b1, rm1, rv1, W2, g2, b2, rm2, rv2, W3, b3):
    feat_t = features.T
    coors_col = coors[:, :3].astype(jnp.bfloat16)
    coors_row = coors_col.T

    full = lambda shape: pl.BlockSpec(shape, lambda i: (0, 0))
    out = pl.pallas_call(
        _body,
        grid=(_N // _BI,),
        in_specs=[
            full((3, _N)),
            full((_N, 3)),
            full((3, _N)),
            full((64, 9)),
            full((64, 1)), full((64, 1)), full((64, 1)), full((64, 1)),
            full((128, 64)),
            full((128, 1)), full((128, 1)), full((128, 1)), full((128, 1)),
            full((128, 128)),
            full((128, 1)),
        ],
        out_specs=pl.BlockSpec((_BI, 128), lambda i: (i, 0)),
        out_shape=jax.ShapeDtypeStruct((_N, 128), jnp.float32),
    )(feat_t, coors_col, coors_row, W1,
      g1.reshape(64, 1), b1.reshape(64, 1), rm1.reshape(64, 1), rv1.reshape(64, 1),
      W2,
      g2.reshape(128, 1), b2.reshape(128, 1), rm2.reshape(128, 1), rv2.reshape(128, 1),
      W3, b3.reshape(128, 1))
    return out
